# baseline (device time: 8393 ns/iter reference)
import os

import jax
import jax.numpy as jnp
from jax import lax
from jax.experimental import pallas as pl
from jax.experimental.pallas import tpu as pltpu

_VARIANT = os.environ.get("KERNEL_VARIANT", "full")

B, SQ, SKV, H, D = 8, 1, 512, 8, 64
NY = 4
NC = 2
CB = B // NC
SCALE = D ** -0.5


def kernel(Q, K, V):
    def body(q_ref, k_hbm, v_hbm, o_ref,
             kbuf, vbuf, comm_ref, dma_sems, send_sems, recv_sems):
        my_x = lax.axis_index("x")
        my_y = lax.axis_index("y")
        my_z = lax.axis_index("z")

        if _VARIANT == "full":
            barrier = pltpu.get_barrier_semaphore()
            for o in (1, 2, 3):
                pl.semaphore_signal(
                    barrier, inc=1,
                    device_id=(my_x, (my_y + o) % NY, my_z),
                    device_id_type=pl.DeviceIdType.MESH,
                )

        copies = []
        for c in range(NC):
            kc = pltpu.make_async_copy(
                k_hbm.at[pl.ds(c * CB, CB)], kbuf.at[c], dma_sems.at[0, c])
            vc = pltpu.make_async_copy(
                v_hbm.at[pl.ds(c * CB, CB)], vbuf.at[c], dma_sems.at[1, c])
            kc.start()
            vc.start()
            copies.append((kc, vc))

        if _VARIANT == "loadonly":
            for kc, vc in copies:
                kc.wait()
                vc.wait()
            o_ref[...] = jnp.concatenate(
                [kbuf[c, :, :, :, 0] + vbuf[c, :, :, :, 0]
                 for c in range(NC)], axis=0).reshape(B, SQ, H, D)
            return

        q3 = q_ref[:, 0, :, :] * SCALE

        rdmas = []
        for c in range(NC):
            kc, vc = copies[c]
            kc.wait()
            vc.wait()
            q3c = q3[c * CB:(c + 1) * CB]
            s = jnp.sum(kbuf[c] * q3c[:, :, :, None], axis=2)
            p = jnp.exp(s)
            den = jnp.sum(p, axis=2)
            num = jnp.sum(vbuf[c] * p[:, :, None, :], axis=3)
            comm_ref[0, c * CB:(c + 1) * CB, :, 0:D] = num
            comm_ref[0, c * CB:(c + 1) * CB, :, D:D + 8] = (
                jnp.broadcast_to(den[:, :, None], (CB, H, 8)))

            if _VARIANT != "full":
                continue
            if c == 0:
                pl.semaphore_wait(barrier, 3)
            for o in (1, 2, 3):
                rdma = pltpu.make_async_remote_copy(
                    src_ref=comm_ref.at[0, pl.ds(c * CB, CB)],
                    dst_ref=comm_ref.at[o, pl.ds(c * CB, CB)],
                    send_sem=send_sems.at[o - 1, c],
                    recv_sem=recv_sems.at[o - 1, c],
                    device_id=(my_x, (my_y + o) % NY, my_z),
                    device_id_type=pl.DeviceIdType.MESH,
                )
                rdma.start()
                rdmas.append(rdma)

        if _VARIANT == "compute":
            tot = comm_ref[0]
            o_ref[...] = (tot[:, :, 0:D] / tot[:, :, D:D + 1]).reshape(
                B, SQ, H, D)
            return

        for rdma in rdmas:
            rdma.wait()

        tot = (comm_ref[0] + comm_ref[1] + comm_ref[2] + comm_ref[3])
        out = tot[:, :, 0:D] / tot[:, :, D:D + 1]
        o_ref[...] = out.reshape(B, SQ, H, D)

    Kt = jnp.transpose(K, (0, 2, 3, 1))
    Vt = jnp.transpose(V, (0, 2, 3, 1))

    return pl.pallas_call(
        body,
        out_shape=jax.ShapeDtypeStruct((B, SQ, H, D), jnp.float32),
        in_specs=[
            pl.BlockSpec(memory_space=pltpu.VMEM),
            pl.BlockSpec(memory_space=pltpu.MemorySpace.HBM),
            pl.BlockSpec(memory_space=pltpu.MemorySpace.HBM),
        ],
        out_specs=pl.BlockSpec(memory_space=pltpu.VMEM),
        scratch_shapes=[
            pltpu.VMEM((NC, CB, H, D, SKV), jnp.float32),
            pltpu.VMEM((NC, CB, H, D, SKV), jnp.float32),
            pltpu.VMEM((NY, B, H, D + 8), jnp.float32),
            pltpu.SemaphoreType.DMA((2, NC)),
            pltpu.SemaphoreType.DMA((3, NC)),
            pltpu.SemaphoreType.DMA((3, NC)),
        ],
        compiler_params=pltpu.CompilerParams(
            collective_id=0 if _VARIANT == "full" else None,
            vmem_limit_bytes=100 * 1024 * 1024,
        ),
    )(Q, Kt, Vt)
